# Initial kernel scaffold; baseline (speedup 1.0000x reference)
import jax
import jax.numpy as jnp
from jax.experimental import pallas as pl


def kernel(x, edge_index, edge_type, W, W_root, b):
    raise NotImplementedError("write your pallas kernel here")


# speedup vs baseline: 22.9906x; 22.9906x over previous
import functools

import jax
import jax.numpy as jnp
from jax import lax
from jax.experimental import pallas as pl
from jax.experimental.pallas import tpu as pltpu
from jax.experimental.pallas import tpu_sc as plsc

_NC = 2
_NS = 16
_L = 16


def _tc_transform(x, w_all):
    n, d = x.shape
    rr = w_all.shape[0]
    nb = 10
    bn = n // nb

    def body(x_ref, w_ref, o_ref):
        o_ref[0] = jnp.dot(x_ref[...], w_ref[0],
                           preferred_element_type=jnp.float32)

    return pl.pallas_call(
        body,
        grid=(rr, nb),
        in_specs=[
            pl.BlockSpec((bn, d), lambda r, b: (b, 0)),
            pl.BlockSpec((1, d, d), lambda r, b: (r, 0, 0)),
        ],
        out_specs=pl.BlockSpec((1, bn, d), lambda r, b: (r, b, 0)),
        out_shape=jax.ShapeDtypeStruct((rr, n, d), jnp.float32),
    )(x, w_all)


def _tc_finish(xw, partial, b, r):
    _, n, d = xw.shape
    nb = 10
    bn = n // nb

    def body(xw_ref, p_ref, b_ref, o_ref):
        t = xw_ref[0] + p_ref[0] + p_ref[1] + b_ref[...]
        o_ref[...] = jnp.where(t >= 0.0, t, 0.2 * t)

    return pl.pallas_call(
        body,
        grid=(nb,),
        in_specs=[
            pl.BlockSpec((1, bn, d), lambda bb: (r, bb, 0)),
            pl.BlockSpec((2, bn, d), lambda bb: (0, bb, 0)),
            pl.BlockSpec((d,), lambda bb: (0,)),
        ],
        out_specs=pl.BlockSpec((bn, d), lambda bb: (bb, 0)),
        out_shape=jax.ShapeDtypeStruct((n, d), jnp.float32),
    )(xw, partial, b)


def _sc_aggregate(xw2, src, dst, et, n, d, r, interpret=False):
    e = src.shape[0]
    nt = _NC * _NS
    ch = 128
    nct = e // ch
    n1, rem1 = nct // _NS, nct % _NS
    n2, rem2 = nct // nt, nct % nt
    npad = ((n + 1279) // 1280) * 1280
    rpt = npad // _NS
    zc = ch
    cpt = (r * n) // _NS
    zcw = 1000
    zw = ((zcw + _L - 1) // _L) * _L

    mesh = plsc.VectorSubcoreMesh(core_axis_name="c", subcore_axis_name="s")

    @functools.partial(
        pl.kernel,
        mesh=mesh,
        out_type=jax.ShapeDtypeStruct((_NC, npad, d), jnp.float32),
        scratch_types=[
            pltpu.VMEM_SHARED((npad, d), jnp.float32),
            pltpu.VMEM_SHARED((r * n,), jnp.float32),
            pltpu.VMEM((ch,), jnp.int32),
            pltpu.VMEM((ch,), jnp.int32),
            pltpu.VMEM((1, ch), jnp.int32),
            pltpu.VMEM((1, ch), jnp.int32),
            pltpu.VMEM((ch,), jnp.int32),
            pltpu.VMEM((ch,), jnp.float32),
            pltpu.VMEM((ch,), jnp.float32),
            pltpu.VMEM((zw,), jnp.float32),
            pltpu.VMEM((ch, d), jnp.float32),
            pltpu.SemaphoreType.DMA,
        ],
    )
    def k(xw_hbm, src_hbm, dst_hbm, et_hbm, out_hbm,
          accum, cnt, s_src, s_et, dst2d, gidx2d, cidx, cntv, ones, zrow,
          rows, sem):
        c = lax.axis_index("c")
        s = lax.axis_index("s")
        gt = c * _NS + s

        def z_rows(i, _):
            for q in range(d // _L):
                rows[i, pl.ds(q * _L, _L)] = jnp.zeros((_L,), jnp.float32)
            return 0
        lax.fori_loop(0, ch, z_rows, 0)

        def z_zrow(g, _):
            zrow[pl.ds(g * _L, _L)] = jnp.zeros((_L,), jnp.float32)
            return 0
        lax.fori_loop(0, zw // _L, z_zrow, 0)

        def f_ones(g, _):
            ones[pl.ds(g * _L, _L)] = jnp.ones((_L,), jnp.float32)
            return 0
        lax.fori_loop(0, ch // _L, f_ones, 0)

        for kk in range(rpt // zc):
            pltpu.sync_copy(rows.at[pl.ds(0, zc)],
                            accum.at[pl.ds(s * rpt + kk * zc, zc)])
        for kk in range(cpt // zcw):
            pltpu.sync_copy(zrow.at[pl.ds(0, zcw)],
                            cnt.at[pl.ds(s * cpt + kk * zcw, zcw)])
        plsc.subcore_barrier()

        def p1(j, _):
            off = (j * _NS + s) * ch
            pltpu.sync_copy(dst_hbm.at[pl.ds(off, ch)], s_src)
            pltpu.sync_copy(et_hbm.at[pl.ds(off, ch)], s_et)

            def cb(g, _):
                sl = pl.ds(g * _L, _L)
                gidx2d[0, sl] = s_et[sl] * n + s_src[sl]
                return 0
            lax.fori_loop(0, ch // _L, cb, 0)
            pltpu.sync_copy(ones, cnt.at[gidx2d.at[0]], add=True)
            return 0
        lax.fori_loop(0, n1 + (s < rem1).astype(jnp.int32), p1, 0)
        plsc.subcore_barrier()

        def p2(j, _):
            off = (j * nt + gt) * ch
            pltpu.sync_copy(src_hbm.at[pl.ds(off, ch)], s_src)
            pltpu.sync_copy(dst_hbm.at[pl.ds(off, ch)], dst2d.at[0])
            pltpu.sync_copy(et_hbm.at[pl.ds(off, ch)], s_et)

            def cb(g, _):
                sl = pl.ds(g * _L, _L)
                ev = s_et[sl]
                gidx2d[0, sl] = ev * n + s_src[sl]
                cidx[sl] = ev * n + dst2d[0, sl]
                return 0
            lax.fori_loop(0, ch // _L, cb, 0)

            pltpu.sync_copy(cnt.at[cidx], cntv)
            pltpu.async_copy(xw_hbm.at[gidx2d.at[0]], rows, sem).wait()

            def mg(g, _):
                cv = cntv[pl.ds(g * _L, _L)]
                scl = 1.0 / jnp.maximum(cv, 1.0)
                for ee in range(_L):
                    srow = jnp.full((_L,), scl[ee], jnp.float32)
                    ri = g * _L + ee
                    for q in range(d // _L):
                        sl = pl.ds(q * _L, _L)
                        rows[ri, sl] = rows[ri, sl] * srow
                return 0
            lax.fori_loop(0, ch // _L, mg, 0)

            pltpu.sync_copy(rows, accum.at[dst2d.at[0]], add=True)
            return 0
        lax.fori_loop(0, n2 + (gt < rem2).astype(jnp.int32), p2, 0)
        plsc.subcore_barrier()

        pltpu.sync_copy(accum.at[pl.ds(s * rpt, rpt)],
                        out_hbm.at[c, pl.ds(s * rpt, rpt)])

    return k(xw2, src, dst, et)


def kernel(x, edge_index, edge_type, W, W_root, b):
    x = x.astype(jnp.float32)
    n, d = x.shape
    r = W.shape[0]
    src = edge_index[0].astype(jnp.int32)
    dst = edge_index[1].astype(jnp.int32)
    et = edge_type.astype(jnp.int32)
    w_all = jnp.concatenate([W.astype(jnp.float32),
                             W_root.astype(jnp.float32)[None]], axis=0)
    xw = _tc_transform(x, w_all)
    partial = _sc_aggregate(xw.reshape((r + 1) * n, d), src, dst, et, n, d, r)
    return _tc_finish(xw, partial, b.astype(jnp.float32), r)


# speedup vs baseline: 23.6038x; 1.0267x over previous
import functools

import jax
import jax.numpy as jnp
from jax import lax
from jax.experimental import pallas as pl
from jax.experimental.pallas import tpu as pltpu
from jax.experimental.pallas import tpu_sc as plsc

_NC = 2
_NS = 16
_L = 16


def _tc_transform(x, w_all):
    n, d = x.shape
    rr = w_all.shape[0]
    nb = 10
    bn = n // nb

    def body(x_ref, w_ref, o_ref):
        o_ref[0] = jnp.dot(x_ref[...], w_ref[0],
                           preferred_element_type=jnp.float32)

    return pl.pallas_call(
        body,
        grid=(rr, nb),
        in_specs=[
            pl.BlockSpec((bn, d), lambda r, b: (b, 0)),
            pl.BlockSpec((1, d, d), lambda r, b: (r, 0, 0)),
        ],
        out_specs=pl.BlockSpec((1, bn, d), lambda r, b: (r, b, 0)),
        out_shape=jax.ShapeDtypeStruct((rr, n, d), jnp.float32),
    )(x, w_all)


def _tc_finish(xw, partial, b, r):
    _, n, d = xw.shape
    nb = 10
    bn = n // nb

    def body(xw_ref, p_ref, b_ref, o_ref):
        t = xw_ref[0] + p_ref[0] + p_ref[1] + b_ref[...]
        o_ref[...] = jnp.where(t >= 0.0, t, 0.2 * t)

    return pl.pallas_call(
        body,
        grid=(nb,),
        in_specs=[
            pl.BlockSpec((1, bn, d), lambda bb: (r, bb, 0)),
            pl.BlockSpec((2, bn, d), lambda bb: (0, bb, 0)),
            pl.BlockSpec((d,), lambda bb: (0,)),
        ],
        out_specs=pl.BlockSpec((bn, d), lambda bb: (bb, 0)),
        out_shape=jax.ShapeDtypeStruct((n, d), jnp.float32),
    )(xw, partial, b)


def _sc_aggregate(xw2, src, dst, et, n, d, r, interpret=False):
    e = src.shape[0]
    nt = _NC * _NS
    ch = 128
    nct = e // ch
    n1, rem1 = nct // _NS, nct % _NS
    n2, rem2 = nct // nt, nct % nt
    npad = ((n + 1279) // 1280) * 1280
    rpt = npad // _NS
    zc = ch
    cpt = (r * n) // _NS
    zcw = 1000
    zw = ((zcw + _L - 1) // _L) * _L

    mesh = plsc.VectorSubcoreMesh(core_axis_name="c", subcore_axis_name="s")

    chb = ch * 4
    rowb = ch * d * 4

    @functools.partial(
        pl.kernel,
        mesh=mesh,
        out_type=jax.ShapeDtypeStruct((_NC, npad, d), jnp.float32),
        scratch_types=[
            pltpu.VMEM_SHARED((npad, d), jnp.float32),
            pltpu.VMEM_SHARED((r * n,), jnp.float32),
            pltpu.VMEM((3, ch), jnp.int32),
            pltpu.VMEM((3, ch), jnp.int32),
            pltpu.VMEM((3, ch), jnp.int32),
            pltpu.VMEM((3, ch), jnp.int32),
            pltpu.VMEM((3, ch), jnp.int32),
            pltpu.VMEM((2, ch), jnp.float32),
            pltpu.VMEM((ch,), jnp.float32),
            pltpu.VMEM((zw,), jnp.float32),
            pltpu.VMEM((2, ch, d), jnp.float32),
            pltpu.SemaphoreType.DMA((3,)),
            pltpu.SemaphoreType.DMA((2,)),
            pltpu.SemaphoreType.DMA((2,)),
            pltpu.SemaphoreType.DMA((2,)),
        ],
    )
    def k(xw_hbm, src_hbm, dst_hbm, et_hbm, out_hbm,
          accum, cnt, srcb, dstb, etb, gidxb, cidxb, cntvb, ones, zrow,
          rows, sem_idx, sem_cnt, sem_row, sem_sca):
        c = lax.axis_index("c")
        s = lax.axis_index("s")
        gt = c * _NS + s

        def z_rows(i, _):
            for q in range(d // _L):
                rows[0, i, pl.ds(q * _L, _L)] = jnp.zeros((_L,), jnp.float32)
            return 0
        lax.fori_loop(0, ch, z_rows, 0)

        def z_zrow(g, _):
            zrow[pl.ds(g * _L, _L)] = jnp.zeros((_L,), jnp.float32)
            return 0
        lax.fori_loop(0, zw // _L, z_zrow, 0)

        def f_ones(g, _):
            ones[pl.ds(g * _L, _L)] = jnp.ones((_L,), jnp.float32)
            return 0
        lax.fori_loop(0, ch // _L, f_ones, 0)

        for kk in range(rpt // zc):
            pltpu.async_copy(rows.at[0, pl.ds(0, zc)],
                             accum.at[pl.ds(s * rpt + kk * zc, zc)],
                             sem_sca.at[0])
        for kk in range(cpt // zcw):
            pltpu.async_copy(zrow.at[pl.ds(0, zcw)],
                             cnt.at[pl.ds(s * cpt + kk * zcw, zcw)],
                             sem_sca.at[0])
        for kk in range(rpt // zc):
            pltpu.make_async_copy(rows.at[0, pl.ds(0, zc)],
                                  accum.at[pl.ds(s * rpt + kk * zc, zc)],
                                  sem_sca.at[0]).wait()
        for kk in range(cpt // zcw):
            pltpu.make_async_copy(zrow.at[pl.ds(0, zcw)],
                                  cnt.at[pl.ds(s * cpt + kk * zcw, zcw)],
                                  sem_sca.at[0]).wait()
        plsc.subcore_barrier()

        nr1 = n1 + (s < rem1).astype(jnp.int32)

        def p1_load(j, u):
            off = (j * _NS + s) * ch
            pltpu.async_copy(dst_hbm.at[pl.ds(off, ch)], dstb.at[u],
                             sem_idx.at[u])
            pltpu.async_copy(et_hbm.at[pl.ds(off, ch)], etb.at[u],
                             sem_idx.at[u])

        def p1_cidx(u):
            def cb(g, _):
                sl = pl.ds(g * _L, _L)
                gidxb[u, sl] = etb[u, sl] * n + dstb[u, sl]
                return 0
            lax.fori_loop(0, ch // _L, cb, 0)

        def p1_wait(j, u):
            off = (j * _NS + s) * ch
            pltpu.make_async_copy(dst_hbm.at[pl.ds(off, ch)], dstb.at[u],
                                  sem_idx.at[u]).wait()
            pltpu.make_async_copy(et_hbm.at[pl.ds(off, ch)], etb.at[u],
                                  sem_idx.at[u]).wait()

        p1_load(0, 0)
        p1_load(1, 1)
        p1_wait(0, 0)
        p1_cidx(0)

        def p1(j, _):
            u0 = j % 3
            u1 = (j + 1) % 3
            u2 = (j + 2) % 3
            p = j % 2
            q = (j + 1) % 2

            @pl.when(j >= 1)
            def _():
                pltpu.make_async_copy(ones, cnt.at[gidxb.at[u2]],
                                      sem_sca.at[q]).wait()

            @pl.when(j + 2 < nr1)
            def _():
                p1_load(j + 2, u2)

            @pl.when(j + 1 < nr1)
            def _():
                p1_wait(j + 1, u1)
                p1_cidx(u1)

            pltpu.async_copy(ones, cnt.at[gidxb.at[u0]], sem_sca.at[p],
                             add=True)
            return 0
        lax.fori_loop(0, nr1, p1, 0)
        pltpu.make_async_copy(ones, cnt.at[gidxb.at[(nr1 + 2) % 3]],
                              sem_sca.at[(nr1 + 1) % 2]).wait()
        plsc.subcore_barrier()

        nr2 = n2 + (gt < rem2).astype(jnp.int32)

        def p2_load(i, u):
            off = (i * nt + gt) * ch
            pltpu.async_copy(src_hbm.at[pl.ds(off, ch)], srcb.at[u],
                             sem_idx.at[u])
            pltpu.async_copy(dst_hbm.at[pl.ds(off, ch)], dstb.at[u],
                             sem_idx.at[u])
            pltpu.async_copy(et_hbm.at[pl.ds(off, ch)], etb.at[u],
                             sem_idx.at[u])

        def p2_idx(u):
            def cb(g, _):
                sl = pl.ds(g * _L, _L)
                ev = etb[u, sl]
                gidxb[u, sl] = ev * n + srcb[u, sl]
                cidxb[u, sl] = ev * n + dstb[u, sl]
                return 0
            lax.fori_loop(0, ch // _L, cb, 0)

        def p2_gather(u, p):
            pltpu.async_copy(cnt.at[cidxb.at[u]], cntvb.at[p], sem_cnt.at[p])
            pltpu.async_copy(xw_hbm.at[gidxb.at[u]], rows.at[p],
                             sem_row.at[p])

        def p2_wait(i, u):
            off = (i * nt + gt) * ch
            pltpu.make_async_copy(src_hbm.at[pl.ds(off, ch)], srcb.at[u],
                                  sem_idx.at[u]).wait()
            pltpu.make_async_copy(dst_hbm.at[pl.ds(off, ch)], dstb.at[u],
                                  sem_idx.at[u]).wait()
            pltpu.make_async_copy(et_hbm.at[pl.ds(off, ch)], etb.at[u],
                                  sem_idx.at[u]).wait()

        p2_load(0, 0)
        p2_load(1, 1)
        p2_wait(0, 0)
        p2_idx(0)
        p2_gather(0, 0)

        def p2(i, _):
            u0 = i % 3
            u1 = (i + 1) % 3
            u2 = (i + 2) % 3
            p = i % 2
            q = (i + 1) % 2

            @pl.when(i >= 1)
            def _():
                pltpu.make_async_copy(rows.at[q], accum.at[dstb.at[u2]],
                                      sem_sca.at[q]).wait()

            @pl.when(i + 2 < nr2)
            def _():
                p2_load(i + 2, u2)

            @pl.when(i + 1 < nr2)
            def _():
                p2_wait(i + 1, u1)
                p2_idx(u1)
                p2_gather(u1, q)

            pltpu.make_async_copy(cnt.at[cidxb.at[u0]], cntvb.at[p],
                                  sem_cnt.at[p]).wait()
            pltpu.make_async_copy(xw_hbm.at[gidxb.at[u0]], rows.at[p],
                                  sem_row.at[p]).wait()

            def mg(g, _):
                cv = cntvb[p, pl.ds(g * _L, _L)]
                scl = 1.0 / jnp.maximum(cv, 1.0)
                for ee in range(_L):
                    srow = jnp.full((_L,), scl[ee], jnp.float32)
                    ri = g * _L + ee
                    for q2 in range(d // _L):
                        sl = pl.ds(q2 * _L, _L)
                        rows[p, ri, sl] = rows[p, ri, sl] * srow
                return 0
            lax.fori_loop(0, ch // _L, mg, 0)

            pltpu.async_copy(rows.at[p], accum.at[dstb.at[u0]],
                             sem_sca.at[p], add=True)
            return 0
        lax.fori_loop(0, nr2, p2, 0)
        pltpu.make_async_copy(rows.at[(nr2 + 1) % 2],
                              accum.at[dstb.at[(nr2 + 2) % 3]],
                              sem_sca.at[(nr2 + 1) % 2]).wait()
        plsc.subcore_barrier()

        pltpu.sync_copy(accum.at[pl.ds(s * rpt, rpt)],
                        out_hbm.at[c, pl.ds(s * rpt, rpt)])

    return k(xw2, src, dst, et)


def kernel(x, edge_index, edge_type, W, W_root, b):
    x = x.astype(jnp.float32)
    n, d = x.shape
    r = W.shape[0]
    src = edge_index[0].astype(jnp.int32)
    dst = edge_index[1].astype(jnp.int32)
    et = edge_type.astype(jnp.int32)
    w_all = jnp.concatenate([W.astype(jnp.float32),
                             W_root.astype(jnp.float32)[None]], axis=0)
    xw = _tc_transform(x, w_all)
    partial = _sc_aggregate(xw.reshape((r + 1) * n, d), src, dst, et, n, d, r)
    return _tc_finish(xw, partial, b.astype(jnp.float32), r)


# speedup vs baseline: 23.6174x; 1.0006x over previous
import functools

import jax
import jax.numpy as jnp
from jax import lax
from jax.experimental import pallas as pl
from jax.experimental.pallas import tpu as pltpu
from jax.experimental.pallas import tpu_sc as plsc

_NC = 2
_NS = 16
_L = 16


def _tc_transform(x, w_all):
    n, d = x.shape
    rr = w_all.shape[0]
    nb = 10
    bn = n // nb

    def body(x_ref, w_ref, o_ref):
        o_ref[0] = jnp.dot(x_ref[...], w_ref[0],
                           preferred_element_type=jnp.float32)

    return pl.pallas_call(
        body,
        grid=(rr, nb),
        in_specs=[
            pl.BlockSpec((bn, d), lambda r, b: (b, 0)),
            pl.BlockSpec((1, d, d), lambda r, b: (r, 0, 0)),
        ],
        out_specs=pl.BlockSpec((1, bn, d), lambda r, b: (r, b, 0)),
        out_shape=jax.ShapeDtypeStruct((rr, n, d), jnp.float32),
    )(x, w_all)


def _tc_finish(xw, partial, b, r):
    _, n, d = xw.shape
    nb = 10
    bn = n // nb

    def body(xw_ref, p_ref, b_ref, o_ref):
        t = xw_ref[0] + p_ref[0] + p_ref[1] + b_ref[...]
        o_ref[...] = jnp.where(t >= 0.0, t, 0.2 * t)

    return pl.pallas_call(
        body,
        grid=(nb,),
        in_specs=[
            pl.BlockSpec((1, bn, d), lambda bb: (r, bb, 0)),
            pl.BlockSpec((2, bn, d), lambda bb: (0, bb, 0)),
            pl.BlockSpec((d,), lambda bb: (0,)),
        ],
        out_specs=pl.BlockSpec((bn, d), lambda bb: (bb, 0)),
        out_shape=jax.ShapeDtypeStruct((n, d), jnp.float32),
    )(xw, partial, b)


def _sc_aggregate(xw2, src, dst, et, n, d, r, interpret=False):
    e = src.shape[0]
    nt = _NC * _NS
    ch = 128
    nct = e // ch
    n1, rem1 = nct // _NS, nct % _NS
    n2, rem2 = nct // nt, nct % nt
    npad = ((n + 1279) // 1280) * 1280
    rpt = npad // _NS
    zc = ch
    cpt = (r * n) // _NS
    zcw = 1000
    zw = ((zcw + _L - 1) // _L) * _L

    mesh = plsc.VectorSubcoreMesh(core_axis_name="c", subcore_axis_name="s")

    chb = ch * 4
    rowb = ch * d * 4

    @functools.partial(
        pl.kernel,
        mesh=mesh,
        out_type=jax.ShapeDtypeStruct((_NC, npad, d), jnp.float32),
        scratch_types=[
            pltpu.VMEM_SHARED((npad, d), jnp.float32),
            pltpu.VMEM_SHARED((r * n,), jnp.float32),
            pltpu.VMEM((3, ch), jnp.int32),
            pltpu.VMEM((3, ch), jnp.int32),
            pltpu.VMEM((3, ch), jnp.int32),
            pltpu.VMEM((3, ch), jnp.int32),
            pltpu.VMEM((3, ch), jnp.int32),
            pltpu.VMEM((2, ch), jnp.float32),
            pltpu.VMEM((ch,), jnp.float32),
            pltpu.VMEM((zw,), jnp.float32),
            pltpu.VMEM((2, ch, d), jnp.float32),
            pltpu.SemaphoreType.DMA((3,)),
            pltpu.SemaphoreType.DMA((2,)),
            pltpu.SemaphoreType.DMA((2,)),
            pltpu.SemaphoreType.DMA((2,)),
        ],
    )
    def k(xw_hbm, src_hbm, dst_hbm, et_hbm, out_hbm,
          accum, cnt, srcb, dstb, etb, gidxb, cidxb, cntvb, ones, zrow,
          rows, sem_idx, sem_cnt, sem_row, sem_sca):
        c = lax.axis_index("c")
        s = lax.axis_index("s")
        gt = c * _NS + s

        def z_rows(i, _):
            for q in range(d // _L):
                rows[0, i, pl.ds(q * _L, _L)] = jnp.zeros((_L,), jnp.float32)
            return 0
        lax.fori_loop(0, ch, z_rows, 0)

        def z_zrow(g, _):
            zrow[pl.ds(g * _L, _L)] = jnp.zeros((_L,), jnp.float32)
            return 0
        lax.fori_loop(0, zw // _L, z_zrow, 0)

        def f_ones(g, _):
            ones[pl.ds(g * _L, _L)] = jnp.ones((_L,), jnp.float32)
            return 0
        lax.fori_loop(0, ch // _L, f_ones, 0)

        for kk in range(rpt // zc):
            pltpu.async_copy(rows.at[0, pl.ds(0, zc)],
                             accum.at[pl.ds(s * rpt + kk * zc, zc)],
                             sem_sca.at[0])
        for kk in range(cpt // zcw):
            pltpu.async_copy(zrow.at[pl.ds(0, zcw)],
                             cnt.at[pl.ds(s * cpt + kk * zcw, zcw)],
                             sem_sca.at[0])
        for kk in range(rpt // zc):
            pltpu.make_async_copy(rows.at[0, pl.ds(0, zc)],
                                  accum.at[pl.ds(s * rpt + kk * zc, zc)],
                                  sem_sca.at[0]).wait()
        for kk in range(cpt // zcw):
            pltpu.make_async_copy(zrow.at[pl.ds(0, zcw)],
                                  cnt.at[pl.ds(s * cpt + kk * zcw, zcw)],
                                  sem_sca.at[0]).wait()
        plsc.subcore_barrier()

        nr1 = n1 + (s < rem1).astype(jnp.int32)

        def p1_load(j, u):
            off = (j * _NS + s) * ch
            pltpu.async_copy(dst_hbm.at[pl.ds(off, ch)], dstb.at[u],
                             sem_idx.at[u])
            pltpu.async_copy(et_hbm.at[pl.ds(off, ch)], etb.at[u],
                             sem_idx.at[u])

        def p1_cidx(u):
            def cb(g, _):
                sl = pl.ds(g * _L, _L)
                gidxb[u, sl] = etb[u, sl] * n + dstb[u, sl]
                return 0
            lax.fori_loop(0, ch // _L, cb, 0)

        def p1_wait(j, u):
            off = (j * _NS + s) * ch
            pltpu.make_async_copy(dst_hbm.at[pl.ds(off, ch)], dstb.at[u],
                                  sem_idx.at[u]).wait()
            pltpu.make_async_copy(et_hbm.at[pl.ds(off, ch)], etb.at[u],
                                  sem_idx.at[u]).wait()

        p1_load(0, 0)
        p1_load(1, 1)
        p1_wait(0, 0)
        p1_cidx(0)

        def p1(j, _):
            u0 = j % 3
            u1 = (j + 1) % 3
            u2 = (j + 2) % 3
            p = j % 2
            q = (j + 1) % 2

            @pl.when(j >= 1)
            def _():
                pltpu.make_async_copy(ones, cnt.at[gidxb.at[u2]],
                                      sem_sca.at[q]).wait()

            @pl.when(j + 2 < nr1)
            def _():
                p1_load(j + 2, u2)

            @pl.when(j + 1 < nr1)
            def _():
                p1_wait(j + 1, u1)
                p1_cidx(u1)

            pltpu.async_copy(ones, cnt.at[gidxb.at[u0]], sem_sca.at[p],
                             add=True)
            return 0
        lax.fori_loop(0, nr1, p1, 0)
        pltpu.make_async_copy(ones, cnt.at[gidxb.at[(nr1 + 2) % 3]],
                              sem_sca.at[(nr1 + 1) % 2]).wait()
        plsc.subcore_barrier()

        nr2 = n2 + (gt < rem2).astype(jnp.int32)

        def p2_load(i, u):
            off = (i * nt + gt) * ch
            pltpu.async_copy(src_hbm.at[pl.ds(off, ch)], srcb.at[u],
                             sem_idx.at[u])
            pltpu.async_copy(dst_hbm.at[pl.ds(off, ch)], dstb.at[u],
                             sem_idx.at[u])
            pltpu.async_copy(et_hbm.at[pl.ds(off, ch)], etb.at[u],
                             sem_idx.at[u])

        def p2_idx(u):
            def cb(g, _):
                sl = pl.ds(g * _L, _L)
                ev = etb[u, sl]
                gidxb[u, sl] = ev * n + srcb[u, sl]
                cidxb[u, sl] = ev * n + dstb[u, sl]
                return 0
            lax.fori_loop(0, ch // _L, cb, 0)

        def p2_gather(u, p):
            pltpu.async_copy(cnt.at[cidxb.at[u]], cntvb.at[p], sem_cnt.at[p])
            pltpu.async_copy(xw_hbm.at[gidxb.at[u]], rows.at[p],
                             sem_row.at[p])

        def p2_wait(i, u):
            off = (i * nt + gt) * ch
            pltpu.make_async_copy(src_hbm.at[pl.ds(off, ch)], srcb.at[u],
                                  sem_idx.at[u]).wait()
            pltpu.make_async_copy(dst_hbm.at[pl.ds(off, ch)], dstb.at[u],
                                  sem_idx.at[u]).wait()
            pltpu.make_async_copy(et_hbm.at[pl.ds(off, ch)], etb.at[u],
                                  sem_idx.at[u]).wait()

        p2_load(0, 0)
        p2_load(1, 1)
        p2_wait(0, 0)
        p2_idx(0)
        p2_gather(0, 0)

        def p2(i, _):
            u0 = i % 3
            u1 = (i + 1) % 3
            u2 = (i + 2) % 3
            p = i % 2
            q = (i + 1) % 2

            @pl.when(i >= 1)
            def _():
                pltpu.make_async_copy(rows.at[q], accum.at[dstb.at[u2]],
                                      sem_sca.at[q]).wait()

            @pl.when(i + 2 < nr2)
            def _():
                p2_load(i + 2, u2)

            @pl.when(i + 1 < nr2)
            def _():
                p2_wait(i + 1, u1)
                p2_idx(u1)
                p2_gather(u1, q)

            pltpu.make_async_copy(cnt.at[cidxb.at[u0]], cntvb.at[p],
                                  sem_cnt.at[p]).wait()
            pltpu.make_async_copy(xw_hbm.at[gidxb.at[u0]], rows.at[p],
                                  sem_row.at[p]).wait()

            def mg(g, _):
                cv = cntvb[p, pl.ds(g * _L, _L)]
                scl = 1.0 / jnp.maximum(cv, 1.0)
                dn = lax.GatherDimensionNumbers(
                    offset_dims=(), collapsed_slice_dims=(0,),
                    start_index_map=(0,))
                for ee in range(_L):
                    srow = lax.gather(
                        scl, jnp.full((_L, 1), ee, jnp.int32), dn, (1,),
                        mode=lax.GatherScatterMode.PROMISE_IN_BOUNDS)
                    ri = g * _L + ee
                    for q2 in range(d // _L):
                        sl = pl.ds(q2 * _L, _L)
                        rows[p, ri, sl] = rows[p, ri, sl] * srow
                return 0
            lax.fori_loop(0, ch // _L, mg, 0)

            pltpu.async_copy(rows.at[p], accum.at[dstb.at[u0]],
                             sem_sca.at[p], add=True)
            return 0
        lax.fori_loop(0, nr2, p2, 0)
        pltpu.make_async_copy(rows.at[(nr2 + 1) % 2],
                              accum.at[dstb.at[(nr2 + 2) % 3]],
                              sem_sca.at[(nr2 + 1) % 2]).wait()
        plsc.subcore_barrier()

        pltpu.sync_copy(accum.at[pl.ds(s * rpt, rpt)],
                        out_hbm.at[c, pl.ds(s * rpt, rpt)])

    return k(xw2, src, dst, et)


def kernel(x, edge_index, edge_type, W, W_root, b):
    x = x.astype(jnp.float32)
    n, d = x.shape
    r = W.shape[0]
    src = edge_index[0].astype(jnp.int32)
    dst = edge_index[1].astype(jnp.int32)
    et = edge_type.astype(jnp.int32)
    w_all = jnp.concatenate([W.astype(jnp.float32),
                             W_root.astype(jnp.float32)[None]], axis=0)
    xw = _tc_transform(x, w_all)
    partial = _sc_aggregate(xw.reshape((r + 1) * n, d), src, dst, et, n, d, r)
    return _tc_finish(xw, partial, b.astype(jnp.float32), r)


# speedup vs baseline: 48.1409x; 2.0384x over previous
import functools

import jax
import jax.numpy as jnp
from jax import lax
from jax.experimental import pallas as pl
from jax.experimental.pallas import tpu as pltpu
from jax.experimental.pallas import tpu_sc as plsc

_NC = 2
_NS = 16
_L = 16


def _tc_transform(x, w_all):
    n, d = x.shape
    rr = w_all.shape[0]
    nb = 10
    bn = n // nb

    def body(x_ref, w_ref, o_ref):
        o_ref[0] = jnp.dot(x_ref[...], w_ref[0],
                           preferred_element_type=jnp.float32)

    return pl.pallas_call(
        body,
        grid=(rr, nb),
        in_specs=[
            pl.BlockSpec((bn, d), lambda r, b: (b, 0)),
            pl.BlockSpec((1, d, d), lambda r, b: (r, 0, 0)),
        ],
        out_specs=pl.BlockSpec((1, bn, d), lambda r, b: (r, b, 0)),
        out_shape=jax.ShapeDtypeStruct((rr, n, d), jnp.float32),
    )(x, w_all)


def _tc_finish(xw, partial, b, r):
    _, n, d = xw.shape
    nb = 10
    bn = n // nb

    def body(xw_ref, p_ref, b_ref, o_ref):
        t = xw_ref[0] + p_ref[0] + p_ref[1] + b_ref[...]
        o_ref[...] = jnp.where(t >= 0.0, t, 0.2 * t)

    return pl.pallas_call(
        body,
        grid=(nb,),
        in_specs=[
            pl.BlockSpec((1, bn, d), lambda bb: (r, bb, 0)),
            pl.BlockSpec((2, bn, d), lambda bb: (0, bb, 0)),
            pl.BlockSpec((d,), lambda bb: (0,)),
        ],
        out_specs=pl.BlockSpec((bn, d), lambda bb: (bb, 0)),
        out_shape=jax.ShapeDtypeStruct((n, d), jnp.float32),
    )(xw, partial, b)


def _sc_aggregate(xw2, src, dst, et, n, d, r, interpret=False):
    e = src.shape[0]
    nt = _NC * _NS
    ch = 128
    nct = e // ch
    n1, rem1 = nct // _NS, nct % _NS
    n2, rem2 = nct // nt, nct % nt
    npad = ((n + 1279) // 1280) * 1280
    rpt = npad // _NS
    zc = ch
    cpt = (r * n) // _NS
    zcw = 1000
    zw = ((zcw + _L - 1) // _L) * _L

    mesh = plsc.VectorSubcoreMesh(core_axis_name="c", subcore_axis_name="s")

    chb = ch * 4
    rowb = ch * d * 4

    @functools.partial(
        pl.kernel,
        mesh=mesh,
        out_type=jax.ShapeDtypeStruct((_NC, npad, d), jnp.float32),
        scratch_types=[
            pltpu.VMEM_SHARED((npad, d), jnp.float32),
            pltpu.VMEM_SHARED((r * n,), jnp.float32),
            pltpu.VMEM((3, ch), jnp.int32),
            pltpu.VMEM((3, ch), jnp.int32),
            pltpu.VMEM((3, ch), jnp.int32),
            pltpu.VMEM((3, ch), jnp.int32),
            pltpu.VMEM((3, ch), jnp.int32),
            pltpu.VMEM((2, ch), jnp.float32),
            pltpu.VMEM((ch,), jnp.float32),
            pltpu.VMEM((zw,), jnp.float32),
            pltpu.VMEM((2, ch, d), jnp.float32),
            pltpu.SemaphoreType.DMA((3,)),
            pltpu.SemaphoreType.DMA((2,)),
            pltpu.SemaphoreType.DMA((2,)),
            pltpu.SemaphoreType.DMA((2,)),
        ],
    )
    def k(xw_hbm, src_hbm, dst_hbm, et_hbm, out_hbm,
          accum, cnt, srcb, dstb, etb, gidxb, cidxb, cntvb, ones, zrow,
          rows, sem_idx, sem_cnt, sem_row, sem_sca):
        c = lax.axis_index("c")
        s = lax.axis_index("s")
        gt = c * _NS + s

        def z_rows(i, _):
            for q in range(d // _L):
                rows[0, i, pl.ds(q * _L, _L)] = jnp.zeros((_L,), jnp.float32)
            return 0
        lax.fori_loop(0, ch, z_rows, 0)

        def z_zrow(g, _):
            zrow[pl.ds(g * _L, _L)] = jnp.zeros((_L,), jnp.float32)
            return 0
        lax.fori_loop(0, zw // _L, z_zrow, 0)

        def f_ones(g, _):
            ones[pl.ds(g * _L, _L)] = jnp.ones((_L,), jnp.float32)
            return 0
        lax.fori_loop(0, ch // _L, f_ones, 0)

        for kk in range(rpt // zc):
            pltpu.async_copy(rows.at[0, pl.ds(0, zc)],
                             accum.at[pl.ds(s * rpt + kk * zc, zc)],
                             sem_sca.at[0])
        for kk in range(cpt // zcw):
            pltpu.async_copy(zrow.at[pl.ds(0, zcw)],
                             cnt.at[pl.ds(s * cpt + kk * zcw, zcw)],
                             sem_sca.at[0])
        for kk in range(rpt // zc):
            pltpu.make_async_copy(rows.at[0, pl.ds(0, zc)],
                                  accum.at[pl.ds(s * rpt + kk * zc, zc)],
                                  sem_sca.at[0]).wait()
        for kk in range(cpt // zcw):
            pltpu.make_async_copy(zrow.at[pl.ds(0, zcw)],
                                  cnt.at[pl.ds(s * cpt + kk * zcw, zcw)],
                                  sem_sca.at[0]).wait()
        plsc.subcore_barrier()

        nr1 = n1 + (s < rem1).astype(jnp.int32)

        def p1_load(j, u):
            off = (j * _NS + s) * ch
            pltpu.async_copy(dst_hbm.at[pl.ds(off, ch)], dstb.at[u],
                             sem_idx.at[u])
            pltpu.async_copy(et_hbm.at[pl.ds(off, ch)], etb.at[u],
                             sem_idx.at[u])

        def p1_cidx(u):
            def cb(g, _):
                sl = pl.ds(g * _L, _L)
                gidxb[u, sl] = etb[u, sl] * n + dstb[u, sl]
                return 0
            lax.fori_loop(0, ch // _L, cb, 0)

        def p1_wait(j, u):
            off = (j * _NS + s) * ch
            pltpu.make_async_copy(dst_hbm.at[pl.ds(off, ch)], dstb.at[u],
                                  sem_idx.at[u]).wait()
            pltpu.make_async_copy(et_hbm.at[pl.ds(off, ch)], etb.at[u],
                                  sem_idx.at[u]).wait()

        p1_load(0, 0)
        p1_load(1, 1)
        p1_wait(0, 0)
        p1_cidx(0)

        def p1(j, _):
            u0 = j % 3
            u1 = (j + 1) % 3
            u2 = (j + 2) % 3
            p = j % 2
            q = (j + 1) % 2

            @pl.when(j >= 1)
            def _():
                pltpu.make_async_copy(ones, cnt.at[gidxb.at[u2]],
                                      sem_sca.at[q]).wait()

            @pl.when(j + 2 < nr1)
            def _():
                p1_load(j + 2, u2)

            @pl.when(j + 1 < nr1)
            def _():
                p1_wait(j + 1, u1)
                p1_cidx(u1)

            pltpu.async_copy(ones, cnt.at[gidxb.at[u0]], sem_sca.at[p],
                             add=True)
            return 0
        lax.fori_loop(0, nr1, p1, 0)
        pltpu.make_async_copy(ones, cnt.at[gidxb.at[(nr1 + 2) % 3]],
                              sem_sca.at[(nr1 + 1) % 2]).wait()
        plsc.subcore_barrier()

        nr2 = n2 + (gt < rem2).astype(jnp.int32)

        def p2_load(i, u):
            off = (i * nt + gt) * ch
            pltpu.async_copy(src_hbm.at[pl.ds(off, ch)], srcb.at[u],
                             sem_idx.at[u])
            pltpu.async_copy(dst_hbm.at[pl.ds(off, ch)], dstb.at[u],
                             sem_idx.at[u])
            pltpu.async_copy(et_hbm.at[pl.ds(off, ch)], etb.at[u],
                             sem_idx.at[u])

        def p2_idx(u):
            def cb(g, _):
                sl = pl.ds(g * _L, _L)
                ev = etb[u, sl]
                gidxb[u, sl] = ev * n + srcb[u, sl]
                cidxb[u, sl] = ev * n + dstb[u, sl]
                return 0
            lax.fori_loop(0, ch // _L, cb, 0)

        def p2_gather(u, p):
            pltpu.async_copy(cnt.at[cidxb.at[u]], cntvb.at[p], sem_cnt.at[p])
            pltpu.async_copy(xw_hbm.at[gidxb.at[u]], rows.at[p],
                             sem_row.at[p])

        def p2_wait(i, u):
            off = (i * nt + gt) * ch
            pltpu.make_async_copy(src_hbm.at[pl.ds(off, ch)], srcb.at[u],
                                  sem_idx.at[u]).wait()
            pltpu.make_async_copy(dst_hbm.at[pl.ds(off, ch)], dstb.at[u],
                                  sem_idx.at[u]).wait()
            pltpu.make_async_copy(et_hbm.at[pl.ds(off, ch)], etb.at[u],
                                  sem_idx.at[u]).wait()

        p2_load(0, 0)
        p2_load(1, 1)
        p2_wait(0, 0)
        p2_idx(0)
        p2_gather(0, 0)

        def p2(i, _):
            u0 = i % 3
            u1 = (i + 1) % 3
            u2 = (i + 2) % 3
            p = i % 2
            q = (i + 1) % 2

            @pl.when(i >= 1)
            def _():
                pltpu.make_async_copy(rows.at[q], accum.at[dstb.at[u2]],
                                      sem_sca.at[q]).wait()

            @pl.when(i + 2 < nr2)
            def _():
                p2_load(i + 2, u2)

            @pl.when(i + 1 < nr2)
            def _():
                p2_wait(i + 1, u1)
                p2_idx(u1)
                p2_gather(u1, q)

            pltpu.make_async_copy(cnt.at[cidxb.at[u0]], cntvb.at[p],
                                  sem_cnt.at[p]).wait()
            pltpu.make_async_copy(xw_hbm.at[gidxb.at[u0]], rows.at[p],
                                  sem_row.at[p]).wait()

            def scpre(g, _):
                sl = pl.ds(g * _L, _L)
                cntvb[p, sl] = 1.0 / jnp.maximum(cntvb[p, sl], 1.0)
                return 0
            lax.fori_loop(0, ch // _L, scpre, 0)

            dn = lax.GatherDimensionNumbers(
                offset_dims=(), collapsed_slice_dims=(0,),
                start_index_map=(0,))

            def mg(ee):
                cv = cntvb[p, pl.ds((ee // _L) * _L, _L)]
                srow = lax.gather(
                    cv, jnp.full((_L, 1), ee % _L, jnp.int32), dn, (1,),
                    mode=lax.GatherScatterMode.PROMISE_IN_BOUNDS)
                vals = [rows[p, ee, pl.ds(q2 * _L, _L)] * srow
                        for q2 in range(d // _L)]
                for q2 in range(d // _L):
                    rows[p, ee, pl.ds(q2 * _L, _L)] = vals[q2]
            plsc.parallel_loop(0, ch, 1, unroll=8)(mg)

            pltpu.async_copy(rows.at[p], accum.at[dstb.at[u0]],
                             sem_sca.at[p], add=True)
            return 0
        lax.fori_loop(0, nr2, p2, 0)
        pltpu.make_async_copy(rows.at[(nr2 + 1) % 2],
                              accum.at[dstb.at[(nr2 + 2) % 3]],
                              sem_sca.at[(nr2 + 1) % 2]).wait()
        plsc.subcore_barrier()

        pltpu.sync_copy(accum.at[pl.ds(s * rpt, rpt)],
                        out_hbm.at[c, pl.ds(s * rpt, rpt)])

    return k(xw2, src, dst, et)


def kernel(x, edge_index, edge_type, W, W_root, b):
    x = x.astype(jnp.float32)
    n, d = x.shape
    r = W.shape[0]
    src = edge_index[0].astype(jnp.int32)
    dst = edge_index[1].astype(jnp.int32)
    et = edge_type.astype(jnp.int32)
    w_all = jnp.concatenate([W.astype(jnp.float32),
                             W_root.astype(jnp.float32)[None]], axis=0)
    xw = _tc_transform(x, w_all)
    partial = _sc_aggregate(xw.reshape((r + 1) * n, d), src, dst, et, n, d, r)
    return _tc_finish(xw, partial, b.astype(jnp.float32), r)


# speedup vs baseline: 57.4744x; 1.1939x over previous
import functools

import jax
import jax.numpy as jnp
from jax import lax
from jax.experimental import pallas as pl
from jax.experimental.pallas import tpu as pltpu
from jax.experimental.pallas import tpu_sc as plsc

_NC = 2
_NS = 16
_L = 16
_CH = 128


def _tc_transform(x, w_all):
    n, d = x.shape
    rr = w_all.shape[0]
    nb = 10
    bn = n // nb

    def body(x_ref, w_ref, o_ref):
        o_ref[0] = jnp.dot(x_ref[...], w_ref[0],
                           preferred_element_type=jnp.float32)

    return pl.pallas_call(
        body,
        grid=(rr, nb),
        in_specs=[
            pl.BlockSpec((bn, d), lambda r, b: (b, 0)),
            pl.BlockSpec((1, d, d), lambda r, b: (r, 0, 0)),
        ],
        out_specs=pl.BlockSpec((1, bn, d), lambda r, b: (r, b, 0)),
        out_shape=jax.ShapeDtypeStruct((rr, n, d), jnp.float32),
    )(x, w_all)


def _tc_finish(xw, partial, b, r):
    _, n, d = xw.shape
    nb = 10
    bn = n // nb

    def body(xw_ref, p_ref, b_ref, o_ref):
        t = xw_ref[0] + p_ref[0] + p_ref[1] + b_ref[...]
        o_ref[...] = jnp.where(t >= 0.0, t, 0.2 * t)

    return pl.pallas_call(
        body,
        grid=(nb,),
        in_specs=[
            pl.BlockSpec((1, bn, d), lambda bb: (r, bb, 0)),
            pl.BlockSpec((2, bn, d), lambda bb: (0, bb, 0)),
            pl.BlockSpec((d,), lambda bb: (0,)),
        ],
        out_specs=pl.BlockSpec((bn, d), lambda bb: (bb, 0)),
        out_shape=jax.ShapeDtypeStruct((n, d), jnp.float32),
    )(xw, partial, b)


def _hist_pad(bins):
    return ((bins + 16 * 32 - 1) // (16 * 32)) * (16 * 32)


def _sc_counts(dst, et, n, r):
    e = dst.shape[0]
    nt = _NC * _NS
    nct = e // _CH
    nc_, remc = nct // nt, nct % nt
    hp = _hist_pad(r * n)
    cpt = hp // _NS
    zcw = cpt // 2

    mesh = plsc.VectorSubcoreMesh(core_axis_name="c", subcore_axis_name="s")
    chb = _CH * 4

    @functools.partial(
        pl.kernel,
        mesh=mesh,
        out_type=jax.ShapeDtypeStruct((_NC * hp,), jnp.float32),
        scratch_types=[
            pltpu.VMEM_SHARED((hp,), jnp.float32),
            pltpu.VMEM((3, _CH), jnp.int32),
            pltpu.VMEM((3, _CH), jnp.int32),
            pltpu.VMEM((3, _CH), jnp.int32),
            pltpu.VMEM((_CH,), jnp.float32),
            pltpu.VMEM((zcw,), jnp.float32),
            pltpu.SemaphoreType.DMA((3,)),
            pltpu.SemaphoreType.DMA((2,)),
        ],
    )
    def k(dst_hbm, et_hbm, out_hbm,
          cnt, dstb, etb, cidxb, ones, zrow, sem_idx, sem_sca):
        c = lax.axis_index("c")
        s = lax.axis_index("s")
        gt = c * _NS + s

        def z_zrow(g, _):
            zrow[pl.ds(g * _L, _L)] = jnp.zeros((_L,), jnp.float32)
            return 0
        lax.fori_loop(0, zcw // _L, z_zrow, 0)

        def f_ones(g, _):
            ones[pl.ds(g * _L, _L)] = jnp.ones((_L,), jnp.float32)
            return 0
        lax.fori_loop(0, _CH // _L, f_ones, 0)

        for kk in range(2):
            pltpu.sync_copy(zrow.at[pl.ds(0, zcw)],
                            cnt.at[pl.ds(s * cpt + kk * zcw, zcw)])
        plsc.subcore_barrier()

        nr = nc_ + (gt < remc).astype(jnp.int32)

        def load(j, u):
            off = (j * nt + gt) * _CH
            pltpu.async_copy(dst_hbm.at[pl.ds(off, _CH)], dstb.at[u],
                             sem_idx.at[u])
            pltpu.async_copy(et_hbm.at[pl.ds(off, _CH)], etb.at[u],
                             sem_idx.at[u])

        def wait_load(j, u):
            off = (j * nt + gt) * _CH
            pltpu.make_async_copy(dst_hbm.at[pl.ds(off, _CH)], dstb.at[u],
                                  sem_idx.at[u]).wait()
            pltpu.make_async_copy(et_hbm.at[pl.ds(off, _CH)], etb.at[u],
                                  sem_idx.at[u]).wait()

        def cidx(u):
            def cb(g, _):
                sl = pl.ds(g * _L, _L)
                cidxb[u, sl] = etb[u, sl] * n + dstb[u, sl]
                return 0
            lax.fori_loop(0, _CH // _L, cb, 0)

        load(0, 0)
        load(1, 1)
        wait_load(0, 0)
        cidx(0)

        def body(j, _):
            u0 = j % 3
            u1 = (j + 1) % 3
            u2 = (j + 2) % 3
            p = j % 2
            q = (j + 1) % 2

            @pl.when(j >= 1)
            def _():
                pltpu.make_async_copy(ones, cnt.at[cidxb.at[u2]],
                                      sem_sca.at[q]).wait()

            @pl.when(j + 2 < nr)
            def _():
                load(j + 2, u2)

            @pl.when(j + 1 < nr)
            def _():
                wait_load(j + 1, u1)
                cidx(u1)

            pltpu.async_copy(ones, cnt.at[cidxb.at[u0]], sem_sca.at[p],
                             add=True)
            return 0
        lax.fori_loop(0, nr, body, 0)
        pltpu.make_async_copy(ones, cnt.at[cidxb.at[(nr + 2) % 3]],
                              sem_sca.at[(nr + 1) % 2]).wait()
        plsc.subcore_barrier()

        for kk in range(2):
            woff = pl.multiple_of(c * hp + s * cpt + kk * zcw, 8)
            pltpu.sync_copy(cnt.at[pl.ds(s * cpt + kk * zcw, zcw)], zrow)
            pltpu.sync_copy(zrow, out_hbm.at[pl.ds(woff, zcw)])

    return k(dst, et)


def _sc_aggregate(xw2, src, dst, et, cntp, n, d, r):
    e = src.shape[0]
    nt = _NC * _NS
    nct = e // _CH
    n2, rem2 = nct // nt, nct % nt
    npad = ((n + 1279) // 1280) * 1280
    rpt = npad // _NS
    zc = _CH
    hp = cntp.shape[0] // _NC
    cpt = hp // _NS
    ccw = cpt // 2

    mesh = plsc.VectorSubcoreMesh(core_axis_name="c", subcore_axis_name="s")
    chb = _CH * 4
    rowb = _CH * d * 4

    @functools.partial(
        pl.kernel,
        mesh=mesh,
        out_type=jax.ShapeDtypeStruct((_NC, npad, d), jnp.float32),
        scratch_types=[
            pltpu.VMEM_SHARED((npad, d), jnp.float32),
            pltpu.VMEM_SHARED((hp,), jnp.float32),
            pltpu.VMEM((3, _CH), jnp.int32),
            pltpu.VMEM((3, _CH), jnp.int32),
            pltpu.VMEM((3, _CH), jnp.int32),
            pltpu.VMEM((3, _CH), jnp.int32),
            pltpu.VMEM((3, _CH), jnp.int32),
            pltpu.VMEM((2, _CH), jnp.float32),
            pltpu.VMEM((ccw,), jnp.float32),
            pltpu.VMEM((ccw,), jnp.float32),
            pltpu.VMEM((2, _CH, d), jnp.float32),
            pltpu.SemaphoreType.DMA((3,)),
            pltpu.SemaphoreType.DMA((2,)),
            pltpu.SemaphoreType.DMA((2,)),
            pltpu.SemaphoreType.DMA((2,)),
        ],
    )
    def k(xw_hbm, src_hbm, dst_hbm, et_hbm, cntp_hbm, out_hbm,
          accum, cnt, srcb, dstb, etb, gidxb, cidxb, cntvb, ca, cb2,
          rows, sem_idx, sem_cnt, sem_row, sem_sca):
        c = lax.axis_index("c")
        s = lax.axis_index("s")
        gt = c * _NS + s

        def z_rows(i, _):
            for q in range(d // _L):
                rows[0, i, pl.ds(q * _L, _L)] = jnp.zeros((_L,), jnp.float32)
            return 0
        lax.fori_loop(0, _CH, z_rows, 0)

        for kk in range(rpt // zc):
            pltpu.async_copy(rows.at[0, pl.ds(0, zc)],
                             accum.at[pl.ds(s * rpt + kk * zc, zc)],
                             sem_sca.at[0])

        for kk in range(2):
            base = pl.multiple_of(s * cpt + kk * ccw, 8)
            pltpu.async_copy(cntp_hbm.at[pl.ds(base, ccw)], ca,
                             sem_cnt.at[0])
            pltpu.async_copy(cntp_hbm.at[pl.ds(hp + base, ccw)], cb2,
                             sem_cnt.at[1])
            pltpu.make_async_copy(cntp_hbm.at[pl.ds(base, ccw)], ca,
                                  sem_cnt.at[0]).wait()
            pltpu.make_async_copy(cntp_hbm.at[pl.ds(hp + base, ccw)], cb2,
                                  sem_cnt.at[1]).wait()

            def addc(g, _):
                sl = pl.ds(g * _L, _L)
                ca[sl] = ca[sl] + cb2[sl]
                return 0
            lax.fori_loop(0, ccw // _L, addc, 0)
            pltpu.sync_copy(ca, cnt.at[pl.ds(base, ccw)])

        for kk in range(rpt // zc):
            pltpu.make_async_copy(rows.at[0, pl.ds(0, zc)],
                                  accum.at[pl.ds(s * rpt + kk * zc, zc)],
                                  sem_sca.at[0]).wait()
        plsc.subcore_barrier()

        nr2 = n2 + (gt < rem2).astype(jnp.int32)

        def p2_load(i, u):
            off = (i * nt + gt) * _CH
            pltpu.async_copy(src_hbm.at[pl.ds(off, _CH)], srcb.at[u],
                             sem_idx.at[u])
            pltpu.async_copy(dst_hbm.at[pl.ds(off, _CH)], dstb.at[u],
                             sem_idx.at[u])
            pltpu.async_copy(et_hbm.at[pl.ds(off, _CH)], etb.at[u],
                             sem_idx.at[u])

        def p2_wait(i, u):
            off = (i * nt + gt) * _CH
            pltpu.make_async_copy(src_hbm.at[pl.ds(off, _CH)], srcb.at[u],
                                  sem_idx.at[u]).wait()
            pltpu.make_async_copy(dst_hbm.at[pl.ds(off, _CH)], dstb.at[u],
                                  sem_idx.at[u]).wait()
            pltpu.make_async_copy(et_hbm.at[pl.ds(off, _CH)], etb.at[u],
                                  sem_idx.at[u]).wait()

        def p2_idx(u):
            def cb(g, _):
                sl = pl.ds(g * _L, _L)
                ev = etb[u, sl]
                gidxb[u, sl] = ev * n + srcb[u, sl]
                cidxb[u, sl] = ev * n + dstb[u, sl]
                return 0
            lax.fori_loop(0, _CH // _L, cb, 0)

        def p2_gather(u, p):
            pltpu.async_copy(cnt.at[cidxb.at[u]], cntvb.at[p], sem_cnt.at[p])
            pltpu.async_copy(xw_hbm.at[gidxb.at[u]], rows.at[p],
                             sem_row.at[p])

        p2_load(0, 0)
        p2_load(1, 1)
        p2_wait(0, 0)
        p2_idx(0)
        p2_gather(0, 0)

        dn = lax.GatherDimensionNumbers(
            offset_dims=(), collapsed_slice_dims=(0,), start_index_map=(0,))

        def p2(i, _):
            u0 = i % 3
            u1 = (i + 1) % 3
            u2 = (i + 2) % 3
            p = i % 2
            q = (i + 1) % 2

            @pl.when(i >= 1)
            def _():
                pltpu.make_async_copy(rows.at[q], accum.at[dstb.at[u2]],
                                      sem_sca.at[q]).wait()

            @pl.when(i + 2 < nr2)
            def _():
                p2_load(i + 2, u2)

            @pl.when(i + 1 < nr2)
            def _():
                p2_wait(i + 1, u1)
                p2_idx(u1)
                p2_gather(u1, q)

            pltpu.make_async_copy(cnt.at[cidxb.at[u0]], cntvb.at[p],
                                  sem_cnt.at[p]).wait()
            pltpu.make_async_copy(xw_hbm.at[gidxb.at[u0]], rows.at[p],
                                  sem_row.at[p]).wait()

            def scpre(g, _):
                sl = pl.ds(g * _L, _L)
                cntvb[p, sl] = 1.0 / jnp.maximum(cntvb[p, sl], 1.0)
                return 0
            lax.fori_loop(0, _CH // _L, scpre, 0)

            def mg(ee):
                cv = cntvb[p, pl.ds((ee // _L) * _L, _L)]
                srow = lax.gather(
                    cv, jnp.full((_L, 1), ee % _L, jnp.int32), dn, (1,),
                    mode=lax.GatherScatterMode.PROMISE_IN_BOUNDS)
                vals = [rows[p, ee, pl.ds(q2 * _L, _L)] * srow
                        for q2 in range(d // _L)]
                for q2 in range(d // _L):
                    rows[p, ee, pl.ds(q2 * _L, _L)] = vals[q2]
            plsc.parallel_loop(0, _CH, 1, unroll=8)(mg)

            pltpu.async_copy(rows.at[p], accum.at[dstb.at[u0]],
                             sem_sca.at[p], add=True)
            return 0
        lax.fori_loop(0, nr2, p2, 0)
        pltpu.make_async_copy(rows.at[(nr2 + 1) % 2],
                              accum.at[dstb.at[(nr2 + 2) % 3]],
                              sem_sca.at[(nr2 + 1) % 2]).wait()
        plsc.subcore_barrier()

        pltpu.sync_copy(accum.at[pl.ds(s * rpt, rpt)],
                        out_hbm.at[c, pl.ds(s * rpt, rpt)])

    return k(xw2, src, dst, et, cntp)


def kernel(x, edge_index, edge_type, W, W_root, b):
    x = x.astype(jnp.float32)
    n, d = x.shape
    r = W.shape[0]
    src = edge_index[0].astype(jnp.int32)
    dst = edge_index[1].astype(jnp.int32)
    et = edge_type.astype(jnp.int32)
    w_all = jnp.concatenate([W.astype(jnp.float32),
                             W_root.astype(jnp.float32)[None]], axis=0)
    xw = _tc_transform(x, w_all)
    cntp = _sc_counts(dst, et, n, r)
    partial = _sc_aggregate(xw.reshape((r + 1) * n, d), src, dst, et, cntp,
                            n, d, r)
    return _tc_finish(xw, partial, b.astype(jnp.float32), r)


# speedup vs baseline: 72.2609x; 1.2573x over previous
import functools

import jax
import jax.numpy as jnp
from jax import lax
from jax.experimental import pallas as pl
from jax.experimental.pallas import tpu as pltpu
from jax.experimental.pallas import tpu_sc as plsc

_NC = 2
_NS = 16
_L = 16
_CH = 128


def _tc_transform(x, w_all):
    n, d = x.shape
    rr = w_all.shape[0]
    nb = 10
    bn = n // nb

    def body(x_ref, w_ref, o_ref):
        for r2 in range(rr):
            o_ref[r2] = jnp.dot(x_ref[...], w_ref[r2],
                                preferred_element_type=jnp.float32)

    return pl.pallas_call(
        body,
        grid=(nb,),
        in_specs=[
            pl.BlockSpec((bn, d), lambda b: (b, 0)),
            pl.BlockSpec((rr, d, d), lambda b: (0, 0, 0)),
        ],
        out_specs=pl.BlockSpec((rr, bn, d), lambda b: (0, b, 0)),
        out_shape=jax.ShapeDtypeStruct((rr, n, d), jnp.float32),
    )(x, w_all)


def _tc_finish(xw, partial, b, r):
    _, n, d = xw.shape
    nb = 10
    bn = n // nb

    def body(xw_ref, p_ref, b_ref, o_ref):
        t = xw_ref[0] + p_ref[0] + p_ref[1] + b_ref[...]
        o_ref[...] = jnp.where(t >= 0.0, t, 0.2 * t)

    return pl.pallas_call(
        body,
        grid=(nb,),
        in_specs=[
            pl.BlockSpec((1, bn, d), lambda bb: (r, bb, 0)),
            pl.BlockSpec((2, bn, d), lambda bb: (0, bb, 0)),
            pl.BlockSpec((d,), lambda bb: (0,)),
        ],
        out_specs=pl.BlockSpec((bn, d), lambda bb: (bb, 0)),
        out_shape=jax.ShapeDtypeStruct((n, d), jnp.float32),
    )(xw, partial, b)


def _hist_pad(bins):
    return ((bins + 16 * 32 - 1) // (16 * 32)) * (16 * 32)


def _sc_counts(eidx, et, e, n, r):
    nt = _NC * _NS
    nct = e // _CH
    nc_, remc = nct // nt, nct % nt
    hp = _hist_pad(r * n)
    cpt = hp // _NS
    zcw = cpt // 2

    mesh = plsc.VectorSubcoreMesh(core_axis_name="c", subcore_axis_name="s")
    chb = _CH * 4

    @functools.partial(
        pl.kernel,
        mesh=mesh,
        out_type=jax.ShapeDtypeStruct((_NC * hp,), jnp.float32),
        scratch_types=[
            pltpu.VMEM_SHARED((hp,), jnp.float32),
            pltpu.VMEM((3, 2, _CH), jnp.int32),
            pltpu.VMEM((3, _CH), jnp.int32),
            pltpu.VMEM((3, _CH), jnp.int32),
            pltpu.VMEM((_CH,), jnp.float32),
            pltpu.VMEM((zcw,), jnp.float32),
            pltpu.SemaphoreType.DMA((3,)),
            pltpu.SemaphoreType.DMA((2,)),
        ],
    )
    def k(eidx_hbm, et_hbm, out_hbm,
          cnt, eb, etb, cidxb, ones, zrow, sem_idx, sem_sca):
        c = lax.axis_index("c")
        s = lax.axis_index("s")
        gt = c * _NS + s

        def z_zrow(g, _):
            zrow[pl.ds(g * _L, _L)] = jnp.zeros((_L,), jnp.float32)
            return 0
        lax.fori_loop(0, zcw // _L, z_zrow, 0)

        def f_ones(g, _):
            ones[pl.ds(g * _L, _L)] = jnp.ones((_L,), jnp.float32)
            return 0
        lax.fori_loop(0, _CH // _L, f_ones, 0)

        for kk in range(2):
            pltpu.sync_copy(zrow.at[pl.ds(0, zcw)],
                            cnt.at[pl.ds(s * cpt + kk * zcw, zcw)])
        plsc.subcore_barrier()

        nr = nc_ + (gt < remc).astype(jnp.int32)

        def load(j, u):
            off = (j * nt + gt) * _CH
            pltpu.async_copy(eidx_hbm.at[:, pl.ds(off, _CH)], eb.at[u],
                             sem_idx.at[u])
            pltpu.async_copy(et_hbm.at[pl.ds(off, _CH)], etb.at[u],
                             sem_idx.at[u])

        def wait_load(j, u):
            off = (j * nt + gt) * _CH
            pltpu.make_async_copy(eidx_hbm.at[:, pl.ds(off, _CH)], eb.at[u],
                                  sem_idx.at[u]).wait()
            pltpu.make_async_copy(et_hbm.at[pl.ds(off, _CH)], etb.at[u],
                                  sem_idx.at[u]).wait()

        def cidx(u):
            def cb(g):
                sl = pl.ds(g * _L, _L)
                cidxb[u, sl] = etb[u, sl] * n + eb[u, 1, sl]
            plsc.parallel_loop(0, _CH // _L, 1, unroll=8)(cb)

        load(0, 0)
        load(1, 1)
        wait_load(0, 0)
        cidx(0)

        def body(j, _):
            u0 = j % 3
            u1 = (j + 1) % 3
            u2 = (j + 2) % 3
            p = j % 2
            q = (j + 1) % 2

            @pl.when(j >= 1)
            def _():
                pltpu.make_async_copy(ones, cnt.at[cidxb.at[u2]],
                                      sem_sca.at[q]).wait()

            @pl.when(j + 2 < nr)
            def _():
                load(j + 2, u2)

            @pl.when(j + 1 < nr)
            def _():
                wait_load(j + 1, u1)
                cidx(u1)

            pltpu.async_copy(ones, cnt.at[cidxb.at[u0]], sem_sca.at[p],
                             add=True)
            return 0
        lax.fori_loop(0, nr, body, 0)
        pltpu.make_async_copy(ones, cnt.at[cidxb.at[(nr + 2) % 3]],
                              sem_sca.at[(nr + 1) % 2]).wait()
        plsc.subcore_barrier()

        for kk in range(2):
            woff = pl.multiple_of(c * hp + s * cpt + kk * zcw, 8)
            pltpu.sync_copy(cnt.at[pl.ds(s * cpt + kk * zcw, zcw)], zrow)
            pltpu.sync_copy(zrow, out_hbm.at[pl.ds(woff, zcw)])

    return k(eidx, et)


def _sc_aggregate(xw2, eidx, et, e, cntp, n, d, r):
    nt = _NC * _NS
    nct = e // _CH
    n2, rem2 = nct // nt, nct % nt
    npad = ((n + 1279) // 1280) * 1280
    rpt = npad // _NS
    zc = _CH
    hp = cntp.shape[0] // _NC
    cpt = hp // _NS
    ccw = cpt // 2

    mesh = plsc.VectorSubcoreMesh(core_axis_name="c", subcore_axis_name="s")
    chb = _CH * 4
    rowb = _CH * d * 4

    @functools.partial(
        pl.kernel,
        mesh=mesh,
        out_type=jax.ShapeDtypeStruct((_NC, npad, d), jnp.float32),
        scratch_types=[
            pltpu.VMEM_SHARED((npad, d), jnp.float32),
            pltpu.VMEM_SHARED((hp,), jnp.float32),
            pltpu.VMEM((3, 2, _CH), jnp.int32),
            pltpu.VMEM((3, _CH), jnp.int32),
            pltpu.VMEM((3, _CH), jnp.int32),
            pltpu.VMEM((3, _CH), jnp.int32),
            pltpu.VMEM((3, _CH), jnp.int32),
            pltpu.VMEM((2, _CH), jnp.float32),
            pltpu.VMEM((ccw,), jnp.float32),
            pltpu.VMEM((ccw,), jnp.float32),
            pltpu.VMEM((2, _CH, d), jnp.float32),
            pltpu.SemaphoreType.DMA((3,)),
            pltpu.SemaphoreType.DMA((2,)),
            pltpu.SemaphoreType.DMA((2,)),
            pltpu.SemaphoreType.DMA((2,)),
        ],
    )
    def k(xw_hbm, eidx_hbm, et_hbm, cntp_hbm, out_hbm,
          accum, cnt, eb, etb, dstb, gidxb, cidxb, cntvb, ca, cb2,
          rows, sem_idx, sem_cnt, sem_row, sem_sca):
        c = lax.axis_index("c")
        s = lax.axis_index("s")
        gt = c * _NS + s

        def z_rows(i, _):
            for q in range(d // _L):
                rows[0, i, pl.ds(q * _L, _L)] = jnp.zeros((_L,), jnp.float32)
            return 0
        lax.fori_loop(0, _CH, z_rows, 0)

        for kk in range(rpt // zc):
            pltpu.async_copy(rows.at[0, pl.ds(0, zc)],
                             accum.at[pl.ds(s * rpt + kk * zc, zc)],
                             sem_sca.at[0])

        for kk in range(2):
            base = pl.multiple_of(s * cpt + kk * ccw, 8)
            pltpu.async_copy(cntp_hbm.at[pl.ds(base, ccw)], ca,
                             sem_cnt.at[0])
            pltpu.async_copy(cntp_hbm.at[pl.ds(hp + base, ccw)], cb2,
                             sem_cnt.at[1])
            pltpu.make_async_copy(cntp_hbm.at[pl.ds(base, ccw)], ca,
                                  sem_cnt.at[0]).wait()
            pltpu.make_async_copy(cntp_hbm.at[pl.ds(hp + base, ccw)], cb2,
                                  sem_cnt.at[1]).wait()

            def addc(g, _):
                sl = pl.ds(g * _L, _L)
                ca[sl] = 1.0 / jnp.maximum(ca[sl] + cb2[sl], 1.0)
                return 0
            lax.fori_loop(0, ccw // _L, addc, 0)
            pltpu.sync_copy(ca, cnt.at[pl.ds(base, ccw)])

        for kk in range(rpt // zc):
            pltpu.make_async_copy(rows.at[0, pl.ds(0, zc)],
                                  accum.at[pl.ds(s * rpt + kk * zc, zc)],
                                  sem_sca.at[0]).wait()
        plsc.subcore_barrier()

        nr2 = n2 + (gt < rem2).astype(jnp.int32)

        def p2_load(i, u):
            off = (i * nt + gt) * _CH
            pltpu.async_copy(eidx_hbm.at[:, pl.ds(off, _CH)], eb.at[u],
                             sem_idx.at[u])
            pltpu.async_copy(et_hbm.at[pl.ds(off, _CH)], etb.at[u],
                             sem_idx.at[u])

        def p2_wait(i, u):
            off = (i * nt + gt) * _CH
            pltpu.make_async_copy(eidx_hbm.at[:, pl.ds(off, _CH)], eb.at[u],
                                  sem_idx.at[u]).wait()
            pltpu.make_async_copy(et_hbm.at[pl.ds(off, _CH)], etb.at[u],
                                  sem_idx.at[u]).wait()

        def p2_idx(u):
            def cb(g):
                sl = pl.ds(g * _L, _L)
                ev = etb[u, sl]
                gidxb[u, sl] = ev * n + eb[u, 0, sl]
                cidxb[u, sl] = ev * n + eb[u, 1, sl]
                dstb[u, sl] = eb[u, 1, sl]
            plsc.parallel_loop(0, _CH // _L, 1, unroll=8)(cb)

        def p2_gather(u, p):
            pltpu.async_copy(cnt.at[cidxb.at[u]], cntvb.at[p], sem_cnt.at[p])
            pltpu.async_copy(xw_hbm.at[gidxb.at[u]], rows.at[p],
                             sem_row.at[p])

        p2_load(0, 0)
        p2_load(1, 1)
        p2_wait(0, 0)
        p2_idx(0)
        p2_gather(0, 0)

        dn = lax.GatherDimensionNumbers(
            offset_dims=(), collapsed_slice_dims=(0,), start_index_map=(0,))

        def p2(i, _):
            u0 = i % 3
            u1 = (i + 1) % 3
            u2 = (i + 2) % 3
            p = i % 2
            q = (i + 1) % 2

            @pl.when(i >= 1)
            def _():
                pltpu.make_async_copy(rows.at[q], accum.at[dstb.at[u2]],
                                      sem_sca.at[q]).wait()

            @pl.when(i + 2 < nr2)
            def _():
                p2_load(i + 2, u2)

            @pl.when(i + 1 < nr2)
            def _():
                p2_wait(i + 1, u1)
                p2_idx(u1)
                p2_gather(u1, q)

            pltpu.make_async_copy(cnt.at[cidxb.at[u0]], cntvb.at[p],
                                  sem_cnt.at[p]).wait()
            pltpu.make_async_copy(xw_hbm.at[gidxb.at[u0]], rows.at[p],
                                  sem_row.at[p]).wait()

            def mg(ee):
                cv = cntvb[p, pl.ds((ee // _L) * _L, _L)]
                srow = lax.gather(
                    cv, jnp.full((_L, 1), ee % _L, jnp.int32), dn, (1,),
                    mode=lax.GatherScatterMode.PROMISE_IN_BOUNDS)
                vals = [rows[p, ee, pl.ds(q2 * _L, _L)] * srow
                        for q2 in range(d // _L)]
                for q2 in range(d // _L):
                    rows[p, ee, pl.ds(q2 * _L, _L)] = vals[q2]
            plsc.parallel_loop(0, _CH, 1, unroll=8)(mg)

            pltpu.async_copy(rows.at[p], accum.at[dstb.at[u0]],
                             sem_sca.at[p], add=True)
            return 0
        lax.fori_loop(0, nr2, p2, 0)
        pltpu.make_async_copy(rows.at[(nr2 + 1) % 2],
                              accum.at[dstb.at[(nr2 + 2) % 3]],
                              sem_sca.at[(nr2 + 1) % 2]).wait()
        plsc.subcore_barrier()

        pltpu.sync_copy(accum.at[pl.ds(s * rpt, rpt)],
                        out_hbm.at[c, pl.ds(s * rpt, rpt)])

    return k(xw2, eidx, et, cntp)


def kernel(x, edge_index, edge_type, W, W_root, b):
    x = x.astype(jnp.float32)
    n, d = x.shape
    r = W.shape[0]
    e = edge_type.shape[0]
    eidx = edge_index.astype(jnp.int32)
    et = edge_type.astype(jnp.int32)
    w_all = jnp.concatenate([W.astype(jnp.float32),
                             W_root.astype(jnp.float32)[None]], axis=0)
    xw = _tc_transform(x, w_all)
    cntp = _sc_counts(eidx, et, e, n, r)
    partial = _sc_aggregate(xw.reshape((r + 1) * n, d), eidx, et, e, cntp,
                            n, d, r)
    return _tc_finish(xw, partial, b.astype(jnp.float32), r)
